# baseline (speedup 1.0000x reference)
import functools

import jax
import jax.numpy as jnp
from jax import lax
from jax.experimental import pallas as pl
from jax.experimental.pallas import tpu as pltpu
from jax.experimental.pallas import tpu_sc as plsc

NC, NS, L = 2, 16, 16
NW = NC * NS

B = 16384
ROW = 128
BPW = B // NW
RPW = 2 * BPW
CHUNK_ROWS = 128
NCHUNK = RPW // CHUNK_ROWS
GROUPS = CHUNK_ROWS // (2 * L)


def _boxes_body(x_hbm, table_hbm, probs_hbm, norm_hbm,
                idx_v, buf_a, buf_b, probs_v, norm_v, sem_a, sem_b):
    wid = lax.axis_index("s") * NC + lax.axis_index("c")

    pltpu.sync_copy(x_hbm.at[wid], idx_v)

    bufs = (buf_a, buf_b)
    sems = (sem_a, sem_b)

    def start(c):
        return pltpu.async_copy(table_hbm.at[idx_v.at[c]], bufs[c % 2], sems[c % 2])

    lanes = lax.iota(jnp.int32, L)
    ones = jnp.ones((L,), jnp.float32)

    copies = [start(0)]
    for c in range(NCHUNK):
        if c + 1 < NCHUNK:
            copies.append(start(c + 1))
        copies[c].wait()
        buf = bufs[c % 2]

        if c == 0:
            @pl.when(wid == 0)
            def _():
                acc = jnp.zeros((L,), jnp.float32)
                for j in range(ROW // L):
                    d0 = buf[2, pl.ds(j * L, L)] - buf[0, pl.ds(j * L, L)]
                    d1 = buf[3, pl.ds(j * L, L)] - buf[1, pl.ds(j * L, L)]
                    acc = acc + d0 * d0 + d1 * d1
                norm_v[...] = acc
                pltpu.sync_copy(norm_v, norm_hbm)

        rows1 = [2 * (g * L + lanes) for g in range(GROUPS)]
        rows2 = [r + 1 for r in rows1]

        def body(d, carry, buf=buf):
            accs = list(carry)
            cmin = (jnp.full((L,), d, jnp.int32) + lanes) & 63
            cmax = cmin + 64
            out = []
            for g in range(GROUPS):
                ai, av = accs[2 * g], accs[2 * g + 1]
                min1 = plsc.load_gather(buf, [rows1[g], cmin])
                max1 = plsc.load_gather(buf, [rows1[g], cmax])
                min2 = plsc.load_gather(buf, [rows2[g], cmin])
                max2 = plsc.load_gather(buf, [rows2[g], cmax])
                e_i = jnp.maximum(
                    jnp.minimum(max1, max2) - jnp.maximum(min1, min2), 0.0)
                e_v = jnp.maximum(max2 - min2, 0.0)
                out.append(ai * e_i)
                out.append(av * e_v)
            return tuple(out)

        accs = lax.fori_loop(0, 64, body, (ones,) * (2 * GROUPS), unroll=2)
        for g in range(GROUPS):
            probs_v[pl.ds(c * (CHUNK_ROWS // 2) + g * L, L)] = (
                accs[2 * g] / accs[2 * g + 1])

    pltpu.sync_copy(probs_v, probs_hbm.at[pl.ds(wid * BPW, BPW)])


@functools.partial(
    pl.kernel,
    out_type=(jax.ShapeDtypeStruct((B,), jnp.float32),
              jax.ShapeDtypeStruct((L,), jnp.float32)),
    mesh=plsc.VectorSubcoreMesh(core_axis_name="c", subcore_axis_name="s"),
    scratch_types=[
        pltpu.VMEM((NCHUNK, CHUNK_ROWS), jnp.int32),
        pltpu.VMEM((CHUNK_ROWS, ROW), jnp.float32),
        pltpu.VMEM((CHUNK_ROWS, ROW), jnp.float32),
        pltpu.VMEM((BPW,), jnp.float32),
        pltpu.VMEM((L,), jnp.float32),
        pltpu.SemaphoreType.DMA,
        pltpu.SemaphoreType.DMA,
    ],
    compiler_params=pltpu.CompilerParams(needs_layout_passes=False),
)
def _boxes_sc(x_hbm, table_hbm, probs_hbm, norm_hbm,
              idx_v, buf_a, buf_b, probs_v, norm_v, sem_a, sem_b):
    _boxes_body(x_hbm, table_hbm, probs_hbm, norm_hbm,
                idx_v, buf_a, buf_b, probs_v, norm_v, sem_a, sem_b)


def kernel(X, boxes):
    num_boxes = boxes.shape[0]
    table = boxes.reshape(num_boxes, ROW)
    x3 = X.astype(jnp.int32).reshape(NW, NCHUNK, CHUNK_ROWS)
    probs, norm16 = _boxes_sc(x3, table)
    norms = jnp.sqrt(jnp.sum(norm16))
    return probs, norms


# speedup vs baseline: 1.1759x; 1.1759x over previous
import functools

import jax
import jax.numpy as jnp
from jax import lax
from jax.experimental import pallas as pl
from jax.experimental.pallas import tpu as pltpu
from jax.experimental.pallas import tpu_sc as plsc

NC, NS, L = 2, 16, 16
NW = NC * NS

B = 16384
ROW = 128
BLK = 128
NBLK = B // BLK
KPW = NBLK // NW
GROUPS = BLK // L
ILV = 4


def _boxes_body(x_hbm, table_hbm, probs_hbm, norm_hbm,
                idx_v, buf1a, buf2a, buf1b, buf2b, probs_v, norm_v,
                sem_a, sem_b):
    wid = lax.axis_index("s") * NC + lax.axis_index("c")

    pltpu.sync_copy(x_hbm.at[pl.ds(wid * KPW, KPW)], idx_v)

    buf1s = (buf1a, buf1b)
    buf2s = (buf2a, buf2b)
    sems = (sem_a, sem_b)

    def start(k):
        p = k % 2
        return (pltpu.async_copy(table_hbm.at[idx_v.at[k, 0]], buf1s[p], sems[p]),
                pltpu.async_copy(table_hbm.at[idx_v.at[k, 1]], buf2s[p], sems[p]))

    lanes = lax.iota(jnp.int32, L)
    ones = jnp.ones((L,), jnp.float32)

    copies = [start(0)]
    for k in range(KPW):
        if k + 1 < KPW:
            copies.append(start(k + 1))
        copies[k][0].wait()
        copies[k][1].wait()
        buf1, buf2 = buf1s[k % 2], buf2s[k % 2]

        if k == 0:
            @pl.when(wid == 0)
            def _():
                acc = jnp.zeros((L,), jnp.float32)
                for j in range(ROW // L):
                    d1 = buf1[1, pl.ds(j * L, L)] - buf1[0, pl.ds(j * L, L)]
                    d2 = buf2[1, pl.ds(j * L, L)] - buf2[0, pl.ds(j * L, L)]
                    acc = acc + d1 * d1 + d2 * d2
                norm_v[...] = acc
                pltpu.sync_copy(norm_v, norm_hbm)

        for g0 in range(0, GROUPS, ILV):
            rows = [g * L + lanes for g in range(g0, g0 + ILV)]

            def body(d, carry, buf1=buf1, buf2=buf2, rows=rows):
                accs = list(carry)
                cmin = (jnp.full((L,), d, jnp.int32) + lanes) & 63
                cmax = cmin + 64
                out = []
                for i in range(ILV):
                    ai, av = accs[2 * i], accs[2 * i + 1]
                    min1 = plsc.load_gather(buf1, [rows[i], cmin])
                    max1 = plsc.load_gather(buf1, [rows[i], cmax])
                    min2 = plsc.load_gather(buf2, [rows[i], cmin])
                    max2 = plsc.load_gather(buf2, [rows[i], cmax])
                    e_i = jnp.maximum(
                        jnp.minimum(max1, max2) - jnp.maximum(min1, min2), 0.0)
                    e_v = jnp.maximum(max2 - min2, 0.0)
                    out.append(ai * e_i)
                    out.append(av * e_v)
                return tuple(out)

            accs = lax.fori_loop(0, 64, body, (ones,) * (2 * ILV), unroll=2)
            for i in range(ILV):
                probs_v[pl.ds(k * BLK + (g0 + i) * L, L)] = (
                    accs[2 * i] / accs[2 * i + 1])

    pltpu.sync_copy(probs_v, probs_hbm.at[pl.ds(wid * KPW * BLK, KPW * BLK)])


@functools.partial(
    pl.kernel,
    out_type=(jax.ShapeDtypeStruct((B,), jnp.float32),
              jax.ShapeDtypeStruct((L,), jnp.float32)),
    mesh=plsc.VectorSubcoreMesh(core_axis_name="c", subcore_axis_name="s"),
    scratch_types=[
        pltpu.VMEM((KPW, 2, BLK), jnp.int32),
        pltpu.VMEM((BLK, ROW), jnp.float32),
        pltpu.VMEM((BLK, ROW), jnp.float32),
        pltpu.VMEM((BLK, ROW), jnp.float32),
        pltpu.VMEM((BLK, ROW), jnp.float32),
        pltpu.VMEM((KPW * BLK,), jnp.float32),
        pltpu.VMEM((L,), jnp.float32),
        pltpu.SemaphoreType.DMA,
        pltpu.SemaphoreType.DMA,
    ],
    compiler_params=pltpu.CompilerParams(needs_layout_passes=False),
)
def _boxes_sc(x_hbm, table_hbm, probs_hbm, norm_hbm,
              idx_v, buf1a, buf2a, buf1b, buf2b, probs_v, norm_v,
              sem_a, sem_b):
    _boxes_body(x_hbm, table_hbm, probs_hbm, norm_hbm,
                idx_v, buf1a, buf2a, buf1b, buf2b, probs_v, norm_v,
                sem_a, sem_b)


def kernel(X, boxes):
    num_boxes = boxes.shape[0]
    table = boxes.reshape(num_boxes, ROW)
    xw = jnp.transpose(X.astype(jnp.int32)).reshape(2, NBLK, BLK)
    xw = jnp.transpose(xw, (1, 0, 2))
    probs, norm16 = _boxes_sc(xw, table)
    norms = jnp.sqrt(jnp.sum(norm16))
    return probs, norms


# speedup vs baseline: 1.1906x; 1.0125x over previous
import functools

import jax
import jax.numpy as jnp
from jax import lax
from jax.experimental import pallas as pl
from jax.experimental.pallas import tpu as pltpu
from jax.experimental.pallas import tpu_sc as plsc

NC, NS, L = 2, 16, 16
NW = NC * NS

B = 16384
ROW = 128
BLK = 128
NBLK = B // BLK
KPW = NBLK // NW
GROUPS = BLK // L
ILV = 4


def _boxes_body(x_hbm, table_hbm, probs_hbm, norm_hbm,
                idx_v, buf1a, buf2a, buf1b, buf2b, probs_v, norm_v,
                sem_a, sem_b):
    wid = lax.axis_index("s") * NC + lax.axis_index("c")

    pltpu.sync_copy(x_hbm.at[pl.ds(wid * KPW, KPW)], idx_v)

    buf1s = (buf1a, buf1b)
    buf2s = (buf2a, buf2b)
    sems = (sem_a, sem_b)

    def start(k):
        p = k % 2
        return (pltpu.async_copy(table_hbm.at[idx_v.at[k, 0]], buf1s[p], sems[p]),
                pltpu.async_copy(table_hbm.at[idx_v.at[k, 1]], buf2s[p], sems[p]))

    lanes = lax.iota(jnp.int32, L)
    ones = jnp.ones((L,), jnp.float32)

    copies = [start(0)]
    for k in range(KPW):
        if k + 1 < KPW:
            copies.append(start(k + 1))
        copies[k][0].wait()
        copies[k][1].wait()
        buf1, buf2 = buf1s[k % 2], buf2s[k % 2]

        if k == 0:
            @pl.when(wid == 0)
            def _():
                acc = jnp.zeros((L,), jnp.float32)
                for j in range(ROW // L):
                    d1 = buf1[1, pl.ds(j * L, L)] - buf1[0, pl.ds(j * L, L)]
                    d2 = buf2[1, pl.ds(j * L, L)] - buf2[0, pl.ds(j * L, L)]
                    acc = acc + d1 * d1 + d2 * d2
                norm_v[...] = acc
                pltpu.sync_copy(norm_v, norm_hbm)

        for g0 in range(0, GROUPS, ILV):
            rows = [g * L + lanes for g in range(g0, g0 + ILV)]

            def body(d, carry, buf1=buf1, buf2=buf2, rows=rows):
                accs = list(carry)
                cmin = (jnp.full((L,), d, jnp.int32) + lanes) & 63
                cmax = cmin + 64
                out = []
                for i in range(ILV):
                    ai, av = accs[2 * i], accs[2 * i + 1]
                    min1 = plsc.load_gather(buf1, [rows[i], cmin])
                    max1 = plsc.load_gather(buf1, [rows[i], cmax])
                    min2 = plsc.load_gather(buf2, [rows[i], cmin])
                    max2 = plsc.load_gather(buf2, [rows[i], cmax])
                    e_i = jnp.maximum(
                        jnp.minimum(max1, max2) - jnp.maximum(min1, min2), 0.0)
                    e_v = jnp.maximum(max2 - min2, 0.0)
                    out.append(ai * e_i)
                    out.append(av * e_v)
                return tuple(out)

            accs = lax.fori_loop(0, 64, body, (ones,) * (2 * ILV), unroll=2)
            for i in range(ILV):
                probs_v[pl.ds(k * BLK + (g0 + i) * L, L)] = (
                    accs[2 * i] / accs[2 * i + 1])

    pltpu.sync_copy(probs_v, probs_hbm.at[pl.ds(wid * KPW * BLK, KPW * BLK)])


@functools.partial(
    pl.kernel,
    out_type=(jax.ShapeDtypeStruct((B,), jnp.float32),
              jax.ShapeDtypeStruct((L,), jnp.float32)),
    mesh=plsc.VectorSubcoreMesh(core_axis_name="c", subcore_axis_name="s"),
    scratch_types=[
        pltpu.VMEM((KPW, 2, BLK), jnp.int32),
        pltpu.VMEM((BLK, ROW), jnp.float32),
        pltpu.VMEM((BLK, ROW), jnp.float32),
        pltpu.VMEM((BLK, ROW), jnp.float32),
        pltpu.VMEM((BLK, ROW), jnp.float32),
        pltpu.VMEM((KPW * BLK,), jnp.float32),
        pltpu.VMEM((L,), jnp.float32),
        pltpu.SemaphoreType.DMA,
        pltpu.SemaphoreType.DMA,
    ],
    compiler_params=pltpu.CompilerParams(needs_layout_passes=False),
)
def _boxes_sc(x_hbm, table_hbm, probs_hbm, norm_hbm,
              idx_v, buf1a, buf2a, buf1b, buf2b, probs_v, norm_v,
              sem_a, sem_b):
    _boxes_body(x_hbm, table_hbm, probs_hbm, norm_hbm,
                idx_v, buf1a, buf2a, buf1b, buf2b, probs_v, norm_v,
                sem_a, sem_b)


_TBN = 3200


def _transpose_body(src, dst):
    dst[...] = src[...].T


def _tc_transpose(xt, num_boxes):
    grid = (num_boxes + _TBN - 1) // _TBN
    return pl.pallas_call(
        _transpose_body,
        grid=(grid,),
        in_specs=[pl.BlockSpec((ROW, _TBN), lambda i: (0, i))],
        out_specs=pl.BlockSpec((_TBN, ROW), lambda i: (i, 0)),
        out_shape=jax.ShapeDtypeStruct((num_boxes, ROW), jnp.float32),
    )(xt)


def kernel(X, boxes):
    num_boxes = boxes.shape[0]
    xt = jnp.transpose(boxes, (1, 2, 0)).reshape(ROW, num_boxes)
    table = _tc_transpose(xt, num_boxes)
    xw = jnp.transpose(X.astype(jnp.int32)).reshape(2, NBLK, BLK)
    xw = jnp.transpose(xw, (1, 0, 2))
    probs, norm16 = _boxes_sc(xw, table)
    norms = jnp.sqrt(jnp.sum(norm16))
    return probs, norms


# speedup vs baseline: 1.3494x; 1.1334x over previous
import functools

import jax
import jax.numpy as jnp
from jax import lax
from jax.experimental import pallas as pl
from jax.experimental.pallas import tpu as pltpu
from jax.experimental.pallas import tpu_sc as plsc

NC, NS, L = 2, 16, 16
NW = NC * NS

B = 16384
ROW = 128
BLK = 128
NBLK = B // BLK
KPW = NBLK // NW
GROUPS = BLK // L
ILV = 4


def _boxes_body(x_hbm, table_hbm, probs_hbm, norm_hbm,
                idx_v, buf1a, buf2a, buf1b, buf2b, probs_v, norm_v,
                sem_a, sem_b):
    wid = lax.axis_index("s") * NC + lax.axis_index("c")

    pltpu.sync_copy(x_hbm.at[pl.ds(wid * KPW, KPW)], idx_v)

    buf1s = (buf1a, buf1b)
    buf2s = (buf2a, buf2b)
    sems = (sem_a, sem_b)

    def start(k):
        p = k % 2
        return (pltpu.async_copy(table_hbm.at[idx_v.at[k, 0]], buf1s[p], sems[p]),
                pltpu.async_copy(table_hbm.at[idx_v.at[k, 1]], buf2s[p], sems[p]))

    lanes = lax.iota(jnp.int32, L)
    ones = jnp.ones((L,), jnp.float32)

    copies = [start(0)]
    for k in range(KPW):
        if k + 1 < KPW:
            copies.append(start(k + 1))
        copies[k][0].wait()
        copies[k][1].wait()
        buf1, buf2 = buf1s[k % 2], buf2s[k % 2]

        if k == 0:
            @pl.when(wid == 0)
            def _():
                acc = jnp.zeros((L,), jnp.float32)
                for j in range(ROW // L):
                    d1 = buf1[1, pl.ds(j * L, L)] - buf1[0, pl.ds(j * L, L)]
                    d2 = buf2[1, pl.ds(j * L, L)] - buf2[0, pl.ds(j * L, L)]
                    acc = acc + d1 * d1 + d2 * d2
                norm_v[...] = acc
                pltpu.sync_copy(norm_v, norm_hbm)

        for g0 in range(0, GROUPS, ILV):
            rows = [g * L + lanes for g in range(g0, g0 + ILV)]

            def body(d, carry, buf1=buf1, buf2=buf2, rows=rows):
                accs = list(carry)
                cmin = (jnp.full((L,), d, jnp.int32) + lanes) & 63
                cmax = cmin + 64
                out = []
                for i in range(ILV):
                    ai, av = accs[2 * i], accs[2 * i + 1]
                    min1 = plsc.load_gather(buf1, [rows[i], cmin])
                    max1 = plsc.load_gather(buf1, [rows[i], cmax])
                    min2 = plsc.load_gather(buf2, [rows[i], cmin])
                    max2 = plsc.load_gather(buf2, [rows[i], cmax])
                    e_i = jnp.maximum(
                        jnp.minimum(max1, max2) - jnp.maximum(min1, min2), 0.0)
                    e_v = jnp.maximum(max2 - min2, 0.0)
                    out.append(ai * e_i)
                    out.append(av * e_v)
                return tuple(out)

            accs = lax.fori_loop(0, 64, body, (ones,) * (2 * ILV), unroll=4)
            for i in range(ILV):
                probs_v[pl.ds(k * BLK + (g0 + i) * L, L)] = (
                    accs[2 * i] / accs[2 * i + 1])

    pltpu.sync_copy(probs_v, probs_hbm.at[pl.ds(wid * KPW * BLK, KPW * BLK)])


@functools.partial(
    pl.kernel,
    out_type=(jax.ShapeDtypeStruct((B,), jnp.float32),
              jax.ShapeDtypeStruct((L,), jnp.float32)),
    mesh=plsc.VectorSubcoreMesh(core_axis_name="c", subcore_axis_name="s"),
    scratch_types=[
        pltpu.VMEM((KPW, 2, BLK), jnp.int32),
        pltpu.VMEM((BLK, ROW), jnp.float32),
        pltpu.VMEM((BLK, ROW), jnp.float32),
        pltpu.VMEM((BLK, ROW), jnp.float32),
        pltpu.VMEM((BLK, ROW), jnp.float32),
        pltpu.VMEM((KPW * BLK,), jnp.float32),
        pltpu.VMEM((L,), jnp.float32),
        pltpu.SemaphoreType.DMA,
        pltpu.SemaphoreType.DMA,
    ],
    compiler_params=pltpu.CompilerParams(needs_layout_passes=False),
)
def _boxes_sc(x_hbm, table_hbm, probs_hbm, norm_hbm,
              idx_v, buf1a, buf2a, buf1b, buf2b, probs_v, norm_v,
              sem_a, sem_b):
    _boxes_body(x_hbm, table_hbm, probs_hbm, norm_hbm,
                idx_v, buf1a, buf2a, buf1b, buf2b, probs_v, norm_v,
                sem_a, sem_b)


_TBN = 25600


def _transpose_body(src, dst):
    dst[...] = src[...].T


def _tc_transpose(xt, num_boxes):
    grid = (num_boxes + _TBN - 1) // _TBN
    return pl.pallas_call(
        _transpose_body,
        grid=(grid,),
        in_specs=[pl.BlockSpec((ROW, _TBN), lambda i: (0, i))],
        out_specs=pl.BlockSpec((_TBN, ROW), lambda i: (i, 0)),
        out_shape=jax.ShapeDtypeStruct((num_boxes, ROW), jnp.float32),
    )(xt)


def kernel(X, boxes):
    num_boxes = boxes.shape[0]
    xt = jnp.transpose(boxes, (1, 2, 0)).reshape(ROW, num_boxes)
    table = _tc_transpose(xt, num_boxes)
    xw = jnp.transpose(X.astype(jnp.int32)).reshape(2, NBLK, BLK)
    xw = jnp.transpose(xw, (1, 0, 2))
    probs, norm16 = _boxes_sc(xw, table)
    norms = jnp.sqrt(jnp.sum(norm16))
    return probs, norms


# speedup vs baseline: 1.3587x; 1.0069x over previous
import functools

import jax
import jax.numpy as jnp
from jax import lax
from jax.experimental import pallas as pl
from jax.experimental.pallas import tpu as pltpu
from jax.experimental.pallas import tpu_sc as plsc

NC, NS, L = 2, 16, 16
NW = NC * NS

B = 16384
ROW = 128
BLK = 128
NBLK = B // BLK
KPW = NBLK // NW
GROUPS = BLK // L
ILV = 8


def _boxes_body(x_hbm, table_hbm, probs_hbm, norm_hbm,
                idx_v, buf1a, buf2a, buf1b, buf2b, probs_v, norm_v,
                sem_a, sem_b):
    wid = lax.axis_index("s") * NC + lax.axis_index("c")

    pltpu.sync_copy(x_hbm.at[pl.ds(wid * KPW, KPW)], idx_v)

    buf1s = (buf1a, buf1b)
    buf2s = (buf2a, buf2b)
    sems = (sem_a, sem_b)

    def start(k):
        p = k % 2
        return (pltpu.async_copy(table_hbm.at[idx_v.at[k, 0]], buf1s[p], sems[p]),
                pltpu.async_copy(table_hbm.at[idx_v.at[k, 1]], buf2s[p], sems[p]))

    lanes = lax.iota(jnp.int32, L)
    ones = jnp.ones((L,), jnp.float32)

    copies = [start(0)]
    for k in range(KPW):
        if k + 1 < KPW:
            copies.append(start(k + 1))
        copies[k][0].wait()
        copies[k][1].wait()
        buf1, buf2 = buf1s[k % 2], buf2s[k % 2]

        if k == 0:
            @pl.when(wid == 0)
            def _():
                acc = jnp.zeros((L,), jnp.float32)
                for j in range(ROW // L):
                    d1 = buf1[1, pl.ds(j * L, L)] - buf1[0, pl.ds(j * L, L)]
                    d2 = buf2[1, pl.ds(j * L, L)] - buf2[0, pl.ds(j * L, L)]
                    acc = acc + d1 * d1 + d2 * d2
                norm_v[...] = acc
                pltpu.sync_copy(norm_v, norm_hbm)

        for g0 in range(0, GROUPS, ILV):
            rows = [g * L + lanes for g in range(g0, g0 + ILV)]

            def body(d, carry, buf1=buf1, buf2=buf2, rows=rows):
                accs = list(carry)
                cmin = (jnp.full((L,), d, jnp.int32) + lanes) & 63
                cmax = cmin + 64
                out = []
                for i in range(ILV):
                    ai, av = accs[2 * i], accs[2 * i + 1]
                    min1 = plsc.load_gather(buf1, [rows[i], cmin])
                    max1 = plsc.load_gather(buf1, [rows[i], cmax])
                    min2 = plsc.load_gather(buf2, [rows[i], cmin])
                    max2 = plsc.load_gather(buf2, [rows[i], cmax])
                    e_i = jnp.maximum(
                        jnp.minimum(max1, max2) - jnp.maximum(min1, min2), 0.0)
                    e_v = jnp.maximum(max2 - min2, 0.0)
                    out.append(ai * e_i)
                    out.append(av * e_v)
                return tuple(out)

            accs = lax.fori_loop(0, 64, body, (ones,) * (2 * ILV), unroll=2)
            for i in range(ILV):
                probs_v[pl.ds(k * BLK + (g0 + i) * L, L)] = (
                    accs[2 * i] / accs[2 * i + 1])

    pltpu.sync_copy(probs_v, probs_hbm.at[pl.ds(wid * KPW * BLK, KPW * BLK)])


@functools.partial(
    pl.kernel,
    out_type=(jax.ShapeDtypeStruct((B,), jnp.float32),
              jax.ShapeDtypeStruct((L,), jnp.float32)),
    mesh=plsc.VectorSubcoreMesh(core_axis_name="c", subcore_axis_name="s"),
    scratch_types=[
        pltpu.VMEM((KPW, 2, BLK), jnp.int32),
        pltpu.VMEM((BLK, ROW), jnp.float32),
        pltpu.VMEM((BLK, ROW), jnp.float32),
        pltpu.VMEM((BLK, ROW), jnp.float32),
        pltpu.VMEM((BLK, ROW), jnp.float32),
        pltpu.VMEM((KPW * BLK,), jnp.float32),
        pltpu.VMEM((L,), jnp.float32),
        pltpu.SemaphoreType.DMA,
        pltpu.SemaphoreType.DMA,
    ],
    compiler_params=pltpu.CompilerParams(needs_layout_passes=False),
)
def _boxes_sc(x_hbm, table_hbm, probs_hbm, norm_hbm,
              idx_v, buf1a, buf2a, buf1b, buf2b, probs_v, norm_v,
              sem_a, sem_b):
    _boxes_body(x_hbm, table_hbm, probs_hbm, norm_hbm,
                idx_v, buf1a, buf2a, buf1b, buf2b, probs_v, norm_v,
                sem_a, sem_b)


_TBN = 25600


def _transpose_body(src, dst):
    dst[...] = src[...].T


def _tc_transpose(xt, num_boxes):
    grid = (num_boxes + _TBN - 1) // _TBN
    return pl.pallas_call(
        _transpose_body,
        grid=(grid,),
        in_specs=[pl.BlockSpec((ROW, _TBN), lambda i: (0, i))],
        out_specs=pl.BlockSpec((_TBN, ROW), lambda i: (i, 0)),
        out_shape=jax.ShapeDtypeStruct((num_boxes, ROW), jnp.float32),
    )(xt)


def kernel(X, boxes):
    num_boxes = boxes.shape[0]
    xt = jnp.transpose(boxes, (1, 2, 0)).reshape(ROW, num_boxes)
    table = _tc_transpose(xt, num_boxes)
    xw = jnp.transpose(X.astype(jnp.int32)).reshape(2, NBLK, BLK)
    xw = jnp.transpose(xw, (1, 0, 2))
    probs, norm16 = _boxes_sc(xw, table)
    norms = jnp.sqrt(jnp.sum(norm16))
    return probs, norms


# speedup vs baseline: 1.3730x; 1.0105x over previous
import functools

import jax
import jax.numpy as jnp
from jax import lax
from jax.experimental import pallas as pl
from jax.experimental.pallas import tpu as pltpu
from jax.experimental.pallas import tpu_sc as plsc

NC, NS, L = 2, 16, 16
NW = NC * NS

B = 16384
ROW = 128
BLK = 128
NBLK = B // BLK
KPW = NBLK // NW
GROUPS = BLK // L
ILV = 4


def _boxes_body(x_hbm, table_hbm, probs_hbm, norm_hbm,
                idx_v, buf1a, buf2a, buf1b, buf2b, buf1c, buf2c,
                probs_v, norm_v, sem_a, sem_b, sem_c):
    wid = lax.axis_index("s") * NC + lax.axis_index("c")

    pltpu.sync_copy(x_hbm.at[pl.ds(wid * KPW, KPW)], idx_v)

    buf1s = (buf1a, buf1b, buf1c)
    buf2s = (buf2a, buf2b, buf2c)
    sems = (sem_a, sem_b, sem_c)

    def start(k):
        p = k % 3
        return (pltpu.async_copy(table_hbm.at[idx_v.at[k, 0]], buf1s[p], sems[p]),
                pltpu.async_copy(table_hbm.at[idx_v.at[k, 1]], buf2s[p], sems[p]))

    lanes = lax.iota(jnp.int32, L)
    ones = jnp.ones((L,), jnp.float32)

    copies = [start(0), start(1)]
    for k in range(KPW):
        if k + 2 < KPW:
            copies.append(start(k + 2))
        copies[k][0].wait()
        copies[k][1].wait()
        buf1, buf2 = buf1s[k % 3], buf2s[k % 3]

        if k == 0:
            @pl.when(wid == 0)
            def _():
                acc = jnp.zeros((L,), jnp.float32)
                for j in range(ROW // L):
                    d1 = buf1[1, pl.ds(j * L, L)] - buf1[0, pl.ds(j * L, L)]
                    d2 = buf2[1, pl.ds(j * L, L)] - buf2[0, pl.ds(j * L, L)]
                    acc = acc + d1 * d1 + d2 * d2
                norm_v[...] = acc
                pltpu.sync_copy(norm_v, norm_hbm)

        for g0 in range(0, GROUPS, ILV):
            rows = [g * L + lanes for g in range(g0, g0 + ILV)]

            def body(d, carry, buf1=buf1, buf2=buf2, rows=rows):
                accs = list(carry)
                cmin = (jnp.full((L,), d, jnp.int32) + lanes) & 63
                cmax = cmin + 64
                out = []
                for i in range(ILV):
                    ai, av = accs[2 * i], accs[2 * i + 1]
                    min1 = plsc.load_gather(buf1, [rows[i], cmin])
                    max1 = plsc.load_gather(buf1, [rows[i], cmax])
                    min2 = plsc.load_gather(buf2, [rows[i], cmin])
                    max2 = plsc.load_gather(buf2, [rows[i], cmax])
                    e_i = jnp.maximum(
                        jnp.minimum(max1, max2) - jnp.maximum(min1, min2), 0.0)
                    e_v = jnp.maximum(max2 - min2, 0.0)
                    out.append(ai * e_i)
                    out.append(av * e_v)
                return tuple(out)

            accs = lax.fori_loop(0, 64, body, (ones,) * (2 * ILV), unroll=2)
            for i in range(ILV):
                probs_v[pl.ds(k * BLK + (g0 + i) * L, L)] = (
                    accs[2 * i] / accs[2 * i + 1])

    pltpu.sync_copy(probs_v, probs_hbm.at[pl.ds(wid * KPW * BLK, KPW * BLK)])


@functools.partial(
    pl.kernel,
    out_type=(jax.ShapeDtypeStruct((B,), jnp.float32),
              jax.ShapeDtypeStruct((L,), jnp.float32)),
    mesh=plsc.VectorSubcoreMesh(core_axis_name="c", subcore_axis_name="s"),
    scratch_types=[
        pltpu.VMEM((KPW, 2, BLK), jnp.int32),
        pltpu.VMEM((BLK, ROW), jnp.float32),
        pltpu.VMEM((BLK, ROW), jnp.float32),
        pltpu.VMEM((BLK, ROW), jnp.float32),
        pltpu.VMEM((BLK, ROW), jnp.float32),
        pltpu.VMEM((BLK, ROW), jnp.float32),
        pltpu.VMEM((BLK, ROW), jnp.float32),
        pltpu.VMEM((KPW * BLK,), jnp.float32),
        pltpu.VMEM((L,), jnp.float32),
        pltpu.SemaphoreType.DMA,
        pltpu.SemaphoreType.DMA,
        pltpu.SemaphoreType.DMA,
    ],
    compiler_params=pltpu.CompilerParams(needs_layout_passes=False),
)
def _boxes_sc(x_hbm, table_hbm, probs_hbm, norm_hbm,
              idx_v, buf1a, buf2a, buf1b, buf2b, buf1c, buf2c,
              probs_v, norm_v, sem_a, sem_b, sem_c):
    _boxes_body(x_hbm, table_hbm, probs_hbm, norm_hbm,
                idx_v, buf1a, buf2a, buf1b, buf2b, buf1c, buf2c,
                probs_v, norm_v, sem_a, sem_b, sem_c)


_TBN = 25600


def _transpose_body(src, dst):
    dst[...] = src[...].T


def _tc_transpose(xt, num_boxes):
    grid = (num_boxes + _TBN - 1) // _TBN
    return pl.pallas_call(
        _transpose_body,
        grid=(grid,),
        in_specs=[pl.BlockSpec((ROW, _TBN), lambda i: (0, i))],
        out_specs=pl.BlockSpec((_TBN, ROW), lambda i: (i, 0)),
        out_shape=jax.ShapeDtypeStruct((num_boxes, ROW), jnp.float32),
    )(xt)


def kernel(X, boxes):
    num_boxes = boxes.shape[0]
    xt = jnp.transpose(boxes, (1, 2, 0)).reshape(ROW, num_boxes)
    table = _tc_transpose(xt, num_boxes)
    xw = jnp.transpose(X.astype(jnp.int32)).reshape(2, NBLK, BLK)
    xw = jnp.transpose(xw, (1, 0, 2))
    probs, norm16 = _boxes_sc(xw, table)
    norms = jnp.sqrt(jnp.sum(norm16))
    return probs, norms


# speedup vs baseline: 1.3781x; 1.0038x over previous
import functools

import jax
import jax.numpy as jnp
from jax import lax
from jax.experimental import pallas as pl
from jax.experimental.pallas import tpu as pltpu
from jax.experimental.pallas import tpu_sc as plsc

NC, NS, L = 2, 16, 16
NW = NC * NS

B = 16384
ROW = 128
BLK = 128
NBLK = B // BLK
KPW = NBLK // NW
GROUPS = BLK // L
ILV = 4


def _boxes_body(x_hbm, table_hbm, probs_hbm, norm_hbm,
                idx_v, buf1a, buf2a, buf1b, buf2b, probs_v, norm_v,
                sem_a, sem_b):
    wid = lax.axis_index("s") * NC + lax.axis_index("c")

    pltpu.sync_copy(x_hbm.at[pl.ds(wid * KPW, KPW)], idx_v)

    buf1s = (buf1a, buf1b)
    buf2s = (buf2a, buf2b)
    sems = (sem_a, sem_b)

    def start(k):
        p = k % 2
        return (pltpu.async_copy(table_hbm.at[idx_v.at[k, 0]], buf1s[p], sems[p]),
                pltpu.async_copy(table_hbm.at[idx_v.at[k, 1]], buf2s[p], sems[p]))

    lanes = lax.iota(jnp.int32, L)
    ones = jnp.ones((L,), jnp.float32)

    copies = [start(0)]
    for k in range(KPW):
        if k + 1 < KPW:
            copies.append(start(k + 1))
        copies[k][0].wait()
        copies[k][1].wait()
        buf1, buf2 = buf1s[k % 2], buf2s[k % 2]

        if k == 0:
            @pl.when(wid == 0)
            def _():
                acc = jnp.zeros((L,), jnp.float32)
                for j in range(ROW // L):
                    d1 = buf1[1, pl.ds(j * L, L)] - buf1[0, pl.ds(j * L, L)]
                    d2 = buf2[1, pl.ds(j * L, L)] - buf2[0, pl.ds(j * L, L)]
                    acc = acc + d1 * d1 + d2 * d2
                norm_v[...] = acc
                pltpu.sync_copy(norm_v, norm_hbm)

        for g0 in range(0, GROUPS, ILV):
            rows = [g * L + lanes for g in range(g0, g0 + ILV)]

            def body(d, carry, buf1=buf1, buf2=buf2, rows=rows):
                accs = list(carry)
                cmin = (jnp.full((L,), d, jnp.int32) + lanes) & 63
                cmax = cmin + 64
                out = []
                for i in range(ILV):
                    ai, av = accs[2 * i], accs[2 * i + 1]
                    min1 = plsc.load_gather(buf1, [rows[i], cmin])
                    max1 = plsc.load_gather(buf1, [rows[i], cmax])
                    min2 = plsc.load_gather(buf2, [rows[i], cmin])
                    max2 = plsc.load_gather(buf2, [rows[i], cmax])
                    e_i = jnp.maximum(
                        jnp.minimum(max1, max2) - jnp.maximum(min1, min2), 0.0)
                    e_v = jnp.maximum(max2 - min2, 0.0)
                    out.append(ai * e_i)
                    out.append(av * e_v)
                return tuple(out)

            accs = lax.fori_loop(0, 64, body, (ones,) * (2 * ILV), unroll=2)
            for i in range(ILV):
                probs_v[pl.ds(k * BLK + (g0 + i) * L, L)] = (
                    accs[2 * i] / accs[2 * i + 1])

    pltpu.sync_copy(probs_v, probs_hbm.at[pl.ds(wid * KPW * BLK, KPW * BLK)])


@functools.partial(
    pl.kernel,
    out_type=(jax.ShapeDtypeStruct((B,), jnp.float32),
              jax.ShapeDtypeStruct((L,), jnp.float32)),
    mesh=plsc.VectorSubcoreMesh(core_axis_name="c", subcore_axis_name="s"),
    scratch_types=[
        pltpu.VMEM((KPW, 2, BLK), jnp.int32),
        pltpu.VMEM((BLK, ROW), jnp.float32),
        pltpu.VMEM((BLK, ROW), jnp.float32),
        pltpu.VMEM((BLK, ROW), jnp.float32),
        pltpu.VMEM((BLK, ROW), jnp.float32),
        pltpu.VMEM((KPW * BLK,), jnp.float32),
        pltpu.VMEM((L,), jnp.float32),
        pltpu.SemaphoreType.DMA,
        pltpu.SemaphoreType.DMA,
    ],
    compiler_params=pltpu.CompilerParams(needs_layout_passes=False),
)
def _boxes_sc(x_hbm, table_hbm, probs_hbm, norm_hbm,
              idx_v, buf1a, buf2a, buf1b, buf2b, probs_v, norm_v,
              sem_a, sem_b):
    _boxes_body(x_hbm, table_hbm, probs_hbm, norm_hbm,
                idx_v, buf1a, buf2a, buf1b, buf2b, probs_v, norm_v,
                sem_a, sem_b)


_TBN = 25600


def _transpose_body(src, dst):
    dst[...] = src[...].T


def _tc_transpose(xt, num_boxes):
    grid = (num_boxes + _TBN - 1) // _TBN
    return pl.pallas_call(
        _transpose_body,
        grid=(grid,),
        in_specs=[pl.BlockSpec((ROW, _TBN), lambda i: (0, i))],
        out_specs=pl.BlockSpec((_TBN, ROW), lambda i: (i, 0)),
        out_shape=jax.ShapeDtypeStruct((num_boxes, ROW), jnp.float32),
    )(xt)


def kernel(X, boxes):
    num_boxes = boxes.shape[0]
    xt = jnp.transpose(boxes, (1, 2, 0)).reshape(ROW, num_boxes)
    table = _tc_transpose(xt, num_boxes)
    xw = jnp.transpose(X.astype(jnp.int32)).reshape(2, NBLK, BLK)
    xw = jnp.transpose(xw, (1, 0, 2))
    probs, norm16 = _boxes_sc(xw, table)
    norms = jnp.sqrt(jnp.sum(norm16))
    return probs, norms
